# baseline (speedup 1.0000x reference)
import functools

import jax
import jax.numpy as jnp
import numpy as np
from jax import lax
from jax.experimental import pallas as pl
from jax.experimental.pallas import tpu as pltpu
from jax.experimental.pallas import tpu_sc as plsc

VOCAB = 100000
HID = 512
BATCH = 4096
SEQ = 50
PAD_IDX = 0

SEQ_PAD = 56
LANES = 16
NC = 2
NS = 16
NW = NC * NS
BPW = BATCH // NW
HG = HID // 32
HST = 8
NBUF = 4

_SIGMA = np.arange(HID).reshape(HG, 16, 2).transpose(0, 2, 1).reshape(HID)


def _bag_kernel(x_hbm, table_hbm, n0_hbm, h_hbm,
                idx_v, buf0_v, buf1_v, buf2_v, buf3_v, hst_v, n0_v, t0_v,
                sem0, sem1, sem2, sem3):
    wid = lax.axis_index("s") * NC + lax.axis_index("c")
    base = pl.multiple_of(wid * BPW, BPW)
    pltpu.sync_copy(x_hbm.at[pl.ds(base, BPW)], idx_v)
    pltpu.sync_copy(n0_hbm.at[pl.ds(base, BPW)], n0_v.at[pl.ds(0, BPW)])
    pltpu.sync_copy(table_hbm.at[pl.ds(0, 16)], t0_v)

    bufs = (buf0_v, buf1_v, buf2_v, buf3_v)
    sems = (sem0, sem1, sem2, sem3)
    for k in range(NBUF):
        pltpu.async_copy(table_hbm.at[idx_v.at[k]], bufs[k], sems[k])

    def process_row(r, buf, sem):
        pltpu.make_async_copy(table_hbm.at[idx_v.at[0]], buf, sem).wait()

        n0f = jnp.broadcast_to(n0_v[pl.ds(r, LANES)][0], (LANES,))
        def unpack2(v):
            a = lax.bitcast_convert_type(lax.shift_left(v, 16), jnp.float32)
            b = lax.bitcast_convert_type(v & jnp.int32(-65536), jnp.float32)
            return a, b

        acc0 = []
        for k in range(HG):
            t0a, t0b = unpack2(t0_v[0, pl.ds(k * LANES, LANES)])
            acc0.append(-n0f * t0a)
            acc0.append(-n0f * t0b)
        acc0 = tuple(acc0)

        def seq_body(j, acc):
            new = []
            for k in range(HG):
                a, b = unpack2(buf[j, pl.ds(k * LANES, LANES)])
                new.append(acc[2 * k] + a)
                new.append(acc[2 * k + 1] + b)
            return tuple(new)

        acc = lax.fori_loop(0, SEQ_PAD, seq_body, acc0, unroll=4)

        @pl.when(r + NBUF < BPW)
        def _():
            pltpu.async_copy(table_hbm.at[idx_v.at[r + NBUF]], buf, sem)

        rr = lax.rem(r, HST)
        for k in range(2 * HG):
            hst_v[rr, pl.ds(k * LANES, LANES)] = acc[k]

        @pl.when(rr == HST - 1)
        def _():
            start = pl.multiple_of(base + r - (HST - 1), HST)
            pltpu.sync_copy(hst_v, h_hbm.at[pl.ds(start, HST)])

    def quad_body(q, carry):
        for b in range(NBUF):
            process_row(NBUF * q + b, bufs[b], sems[b])
        return carry

    lax.fori_loop(0, BPW // NBUF, quad_body, 0)


def _bag(xp, tableb, n0f):
    mesh = plsc.VectorSubcoreMesh(core_axis_name="c", subcore_axis_name="s")
    kern = functools.partial(
        pl.kernel,
        out_type=jax.ShapeDtypeStruct((BATCH, HID), jnp.float32),
        mesh=mesh,
        scratch_types=[
            pltpu.VMEM((BPW, SEQ_PAD), jnp.int32),
            pltpu.VMEM((SEQ_PAD, HID // 2), jnp.int32),
            pltpu.VMEM((SEQ_PAD, HID // 2), jnp.int32),
            pltpu.VMEM((SEQ_PAD, HID // 2), jnp.int32),
            pltpu.VMEM((SEQ_PAD, HID // 2), jnp.int32),
            pltpu.VMEM((HST, HID), jnp.float32),
            pltpu.VMEM((BPW + LANES,), jnp.float32),
            pltpu.VMEM((16, HID // 2), jnp.int32),
            pltpu.SemaphoreType.DMA,
            pltpu.SemaphoreType.DMA,
            pltpu.SemaphoreType.DMA,
            pltpu.SemaphoreType.DMA,
        ],
    )(_bag_kernel)
    return kern(xp, tableb, n0f)


MLP_BB = 512


def _mlp_body(h_ref, w1_ref, b1_ref, w2_ref, b2_ref, out1_ref, out2_ref):
    dn = (((1,), (1,)), ((), ()))
    h = h_ref[...]
    h1 = jnp.maximum(
        lax.dot_general(h, w1_ref[...], dn,
                        preferred_element_type=jnp.float32) + b1_ref[...], 0.0)
    h2 = jnp.maximum(
        lax.dot_general(h1, w2_ref[...], dn,
                        preferred_element_type=jnp.float32) + b2_ref[...], 0.0)
    out1_ref[0] = h2
    out2_ref[0] = h1
    out2_ref[1] = h2


def _mlp(h, W1s, b1, W2, b2):
    out1, out2 = pl.pallas_call(
        _mlp_body,
        grid=(BATCH // MLP_BB,),
        in_specs=[
            pl.BlockSpec((MLP_BB, HID), lambda i: (i, 0)),
            pl.BlockSpec((HID, HID), lambda i: (0, 0)),
            pl.BlockSpec((1, HID), lambda i: (0, 0)),
            pl.BlockSpec((HID, HID), lambda i: (0, 0)),
            pl.BlockSpec((1, HID), lambda i: (0, 0)),
        ],
        out_specs=[
            pl.BlockSpec((1, MLP_BB, HID), lambda i: (0, i, 0)),
            pl.BlockSpec((2, MLP_BB, HID), lambda i: (0, i, 0)),
        ],
        out_shape=[
            jax.ShapeDtypeStruct((1, BATCH, HID), jnp.float32),
            jax.ShapeDtypeStruct((2, BATCH, HID), jnp.float32),
        ],
    )(h, W1s, b1.reshape(1, HID), W2, b2.reshape(1, HID))
    return out1, out2


def kernel(x, table, W1, b1, W2, b2):
    xp = jnp.pad(x.astype(jnp.int32), ((0, 0), (0, SEQ_PAD - SEQ)))
    n0f = jnp.sum((xp == 0).astype(jnp.float32), axis=1)
    tablei = lax.bitcast_convert_type(
        table.astype(jnp.bfloat16).reshape(VOCAB, HID // 2, 2), jnp.int32)
    W1s = jnp.take(W1, jnp.asarray(_SIGMA), axis=1)
    h = _bag(xp, tablei, n0f)
    out1, out2 = _mlp(h, W1s, b1, W2, b2)
    return (out1, out2)


# speedup vs baseline: 1.7969x; 1.7969x over previous
import functools

import jax
import jax.numpy as jnp
from jax import lax
from jax.experimental import pallas as pl
from jax.experimental.pallas import tpu as pltpu
from jax.experimental.pallas import tpu_sc as plsc

VOCAB = 100000
HID = 512
BATCH = 4096
SEQ = 50
PAD_IDX = 0

SEQ_PAD = 56
LANES = 16
NC = 2
NS = 16
NW = NC * NS
BPW = BATCH // NW
HC = HID // LANES
HST = 8
NBUF = 3
NFULL = (BPW // NBUF) * NBUF


def _bag_kernel(x_hbm, table_hbm, n0_hbm, h_hbm,
                idx_v, buf0_v, buf1_v, buf2_v, hst_v, n0_v, t0_v,
                sem0, sem1, sem2):
    wid = lax.axis_index("s") * NC + lax.axis_index("c")
    base = pl.multiple_of(wid * BPW, BPW)
    pltpu.sync_copy(x_hbm.at[pl.ds(base, BPW)], idx_v)
    pltpu.sync_copy(n0_hbm.at[pl.ds(base, BPW)], n0_v.at[pl.ds(0, BPW)])
    pltpu.sync_copy(table_hbm.at[0], t0_v)

    bufs = (buf0_v, buf1_v, buf2_v)
    sems = (sem0, sem1, sem2)
    for k in range(NBUF):
        pltpu.async_copy(table_hbm.at[idx_v.at[k]], bufs[k], sems[k])

    def process_row(r, buf, sem):
        pltpu.make_async_copy(table_hbm.at[idx_v.at[0]], buf, sem).wait()

        n0f = jnp.broadcast_to(n0_v[pl.ds(r, LANES)][0], (LANES,))
        acc0 = tuple(-n0f * t0_v[pl.ds(c * LANES, LANES)]
                     for c in range(HC))

        def seq_body(j, acc):
            return tuple(acc[c] + buf[j, pl.ds(c * LANES, LANES)]
                         for c in range(HC))

        acc = lax.fori_loop(0, SEQ_PAD, seq_body, acc0, unroll=4)

        @pl.when(r + NBUF < BPW)
        def _():
            pltpu.async_copy(table_hbm.at[idx_v.at[r + NBUF]], buf, sem)

        rr = lax.rem(r, HST)
        for c in range(HC):
            hst_v[rr, pl.ds(c * LANES, LANES)] = acc[c]

        @pl.when(rr == HST - 1)
        def _():
            start = pl.multiple_of(base + r - (HST - 1), HST)
            pltpu.sync_copy(hst_v, h_hbm.at[pl.ds(start, HST)])

    def tri_body(q, carry):
        for b in range(NBUF):
            process_row(NBUF * q + b, bufs[b], sems[b])
        return carry

    lax.fori_loop(0, NFULL // NBUF, tri_body, 0)
    for r in range(NFULL, BPW):
        process_row(r, bufs[r % NBUF], sems[r % NBUF])


def _bag(xp, table, n0f):
    mesh = plsc.VectorSubcoreMesh(core_axis_name="c", subcore_axis_name="s")
    kern = functools.partial(
        pl.kernel,
        out_type=jax.ShapeDtypeStruct((BATCH, HID), jnp.float32),
        mesh=mesh,
        scratch_types=[
            pltpu.VMEM((BPW, SEQ_PAD), jnp.int32),
            pltpu.VMEM((SEQ_PAD, HID), jnp.float32),
            pltpu.VMEM((SEQ_PAD, HID), jnp.float32),
            pltpu.VMEM((SEQ_PAD, HID), jnp.float32),
            pltpu.VMEM((HST, HID), jnp.float32),
            pltpu.VMEM((BPW + LANES,), jnp.float32),
            pltpu.VMEM((HID,), jnp.float32),
            pltpu.SemaphoreType.DMA,
            pltpu.SemaphoreType.DMA,
            pltpu.SemaphoreType.DMA,
        ],
    )(_bag_kernel)
    return kern(xp, table, n0f)


MLP_BB = 512


def _mlp_body(h_ref, w1_ref, b1_ref, w2_ref, b2_ref, out1_ref, out2_ref):
    dn = (((1,), (1,)), ((), ()))
    h = h_ref[...]
    h1 = jnp.maximum(
        lax.dot_general(h, w1_ref[...], dn,
                        preferred_element_type=jnp.float32) + b1_ref[...], 0.0)
    h2 = jnp.maximum(
        lax.dot_general(h1, w2_ref[...], dn,
                        preferred_element_type=jnp.float32) + b2_ref[...], 0.0)
    out1_ref[0] = h2
    out2_ref[0] = h1
    out2_ref[1] = h2


def _mlp(h, W1, b1, W2, b2):
    out1, out2 = pl.pallas_call(
        _mlp_body,
        grid=(BATCH // MLP_BB,),
        in_specs=[
            pl.BlockSpec((MLP_BB, HID), lambda i: (i, 0)),
            pl.BlockSpec((HID, HID), lambda i: (0, 0)),
            pl.BlockSpec((1, HID), lambda i: (0, 0)),
            pl.BlockSpec((HID, HID), lambda i: (0, 0)),
            pl.BlockSpec((1, HID), lambda i: (0, 0)),
        ],
        out_specs=[
            pl.BlockSpec((1, MLP_BB, HID), lambda i: (0, i, 0)),
            pl.BlockSpec((2, MLP_BB, HID), lambda i: (0, i, 0)),
        ],
        out_shape=[
            jax.ShapeDtypeStruct((1, BATCH, HID), jnp.float32),
            jax.ShapeDtypeStruct((2, BATCH, HID), jnp.float32),
        ],
    )(h, W1, b1.reshape(1, HID), W2, b2.reshape(1, HID))
    return out1, out2


def kernel(x, table, W1, b1, W2, b2):
    xp = jnp.pad(x.astype(jnp.int32), ((0, 0), (0, SEQ_PAD - SEQ)))
    n0f = jnp.sum((xp == 0).astype(jnp.float32), axis=1)
    h = _bag(xp, table, n0f)
    out1, out2 = _mlp(h, W1, b1, W2, b2)
    return (out1, out2)


# speedup vs baseline: 5.8947x; 3.2806x over previous
import functools

import jax
import jax.numpy as jnp
from jax import lax
from jax.experimental import pallas as pl
from jax.experimental.pallas import tpu as pltpu
from jax.experimental.pallas import tpu_sc as plsc

VOCAB = 100000
HID = 512
BATCH = 4096
SEQ = 50
PAD_IDX = 0

SEQ_PAD = 56
LANES = 16
NC = 2
NS = 16
NW = NC * NS
BPW = BATCH // NW
HC = HID // LANES
HST = 8
NBUF = 3
NFULL = (BPW // NBUF) * NBUF


def _bag_kernel(x_hbm, table_hbm, n0_hbm, h_hbm,
                idx_v, buf0_v, buf1_v, buf2_v, hst_v, n0_v, t0_v,
                sem0, sem1, sem2):
    wid = lax.axis_index("s") * NC + lax.axis_index("c")
    base = pl.multiple_of(wid * BPW, BPW)
    pltpu.sync_copy(x_hbm.at[pl.ds(base, BPW)], idx_v)
    pltpu.sync_copy(n0_hbm.at[pl.ds(base, BPW)], n0_v.at[pl.ds(0, BPW)])
    pltpu.sync_copy(table_hbm.at[0], t0_v)

    bufs = (buf0_v, buf1_v, buf2_v)
    sems = (sem0, sem1, sem2)
    for k in range(NBUF):
        pltpu.async_copy(table_hbm.at[idx_v.at[k]], bufs[k], sems[k])

    def process_row(r, buf, sem):
        pltpu.make_async_copy(table_hbm.at[idx_v.at[0]], buf, sem).wait()

        n0f = jnp.broadcast_to(n0_v[pl.ds(r, LANES)][0], (LANES,))
        acc0 = tuple(-n0f * t0_v[pl.ds(c * LANES, LANES)]
                     for c in range(HC))

        def seq_body(j, acc):
            return tuple(acc[c] + buf[j, pl.ds(c * LANES, LANES)]
                         for c in range(HC))

        acc = lax.fori_loop(0, SEQ, seq_body, acc0, unroll=5)

        @pl.when(r + NBUF < BPW)
        def _():
            pltpu.async_copy(table_hbm.at[idx_v.at[r + NBUF]], buf, sem)

        rr = lax.rem(r, HST)
        for c in range(HC):
            hst_v[rr, pl.ds(c * LANES, LANES)] = acc[c]

        @pl.when(rr == HST - 1)
        def _():
            start = pl.multiple_of(base + r - (HST - 1), HST)
            pltpu.sync_copy(hst_v, h_hbm.at[pl.ds(start, HST)])

    def tri_body(q, carry):
        for b in range(NBUF):
            process_row(NBUF * q + b, bufs[b], sems[b])
        return carry

    lax.fori_loop(0, NFULL // NBUF, tri_body, 0)
    for r in range(NFULL, BPW):
        process_row(r, bufs[r % NBUF], sems[r % NBUF])


def _bag(xp, table, n0f):
    mesh = plsc.VectorSubcoreMesh(core_axis_name="c", subcore_axis_name="s")
    kern = functools.partial(
        pl.kernel,
        out_type=jax.ShapeDtypeStruct((BATCH, HID), jnp.float32),
        mesh=mesh,
        scratch_types=[
            pltpu.VMEM((BPW, SEQ_PAD), jnp.int32),
            pltpu.VMEM((SEQ_PAD, HID), jnp.float32),
            pltpu.VMEM((SEQ_PAD, HID), jnp.float32),
            pltpu.VMEM((SEQ_PAD, HID), jnp.float32),
            pltpu.VMEM((HST, HID), jnp.float32),
            pltpu.VMEM((BPW + LANES,), jnp.float32),
            pltpu.VMEM((HID,), jnp.float32),
            pltpu.SemaphoreType.DMA,
            pltpu.SemaphoreType.DMA,
            pltpu.SemaphoreType.DMA,
        ],
    )(_bag_kernel)
    return kern(xp, table, n0f)


MLP_BB = 512


def _mlp_body(h_ref, w1_ref, b1_ref, w2_ref, b2_ref, out1_ref, out2_ref):
    dn = (((1,), (1,)), ((), ()))
    h = h_ref[...]
    h1 = jnp.maximum(
        lax.dot_general(h, w1_ref[...], dn,
                        preferred_element_type=jnp.float32) + b1_ref[...], 0.0)
    h2 = jnp.maximum(
        lax.dot_general(h1, w2_ref[...], dn,
                        preferred_element_type=jnp.float32) + b2_ref[...], 0.0)
    out1_ref[0] = h2
    out2_ref[0] = h1
    out2_ref[1] = h2


def _mlp(h, W1, b1, W2, b2):
    out1, out2 = pl.pallas_call(
        _mlp_body,
        grid=(BATCH // MLP_BB,),
        in_specs=[
            pl.BlockSpec((MLP_BB, HID), lambda i: (i, 0)),
            pl.BlockSpec((HID, HID), lambda i: (0, 0)),
            pl.BlockSpec((1, HID), lambda i: (0, 0)),
            pl.BlockSpec((HID, HID), lambda i: (0, 0)),
            pl.BlockSpec((1, HID), lambda i: (0, 0)),
        ],
        out_specs=[
            pl.BlockSpec((1, MLP_BB, HID), lambda i: (0, i, 0)),
            pl.BlockSpec((2, MLP_BB, HID), lambda i: (0, i, 0)),
        ],
        out_shape=[
            jax.ShapeDtypeStruct((1, BATCH, HID), jnp.float32),
            jax.ShapeDtypeStruct((2, BATCH, HID), jnp.float32),
        ],
    )(h, W1, b1.reshape(1, HID), W2, b2.reshape(1, HID))
    return out1, out2


def kernel(x, table, W1, b1, W2, b2):
    x32 = x.astype(jnp.int32)
    xp = jnp.concatenate([x32, x32[:, SEQ - (SEQ_PAD - SEQ):]], axis=1)
    n0f = jnp.sum((x32 == 0).astype(jnp.float32), axis=1)
    h = _bag(xp, table, n0f)
    out1, out2 = _mlp(h, W1, b1, W2, b2)
    return (out1, out2)
